# baseline (speedup 1.0000x reference)
import functools

import jax
import jax.numpy as jnp
from jax import lax
from jax.experimental import pallas as pl
from jax.experimental.pallas import tpu as pltpu
from jax.experimental.pallas import tpu_sc as plsc

N = 1_000_000
NIDX = 65536
NW = 32
CHUNK = 31360
LAST = N - 31 * CHUNK
L = 16


@functools.partial(
    pl.kernel,
    mesh=plsc.VectorSubcoreMesh(core_axis_name="c", subcore_axis_name="s",
                                num_cores=2),
    out_type=jax.ShapeDtypeStruct((N,), jnp.float32),
    scratch_types=[
        pltpu.VMEM((NIDX,), jnp.int32),
        pltpu.VMEM((CHUNK,), jnp.float32),
        [pltpu.SemaphoreType.DMA] * 8,
    ],
    compiler_params=pltpu.CompilerParams(needs_layout_passes=False),
)
def _scatter_ones(idx_hbm, out_hbm, idx_v, chunk_v, sems):
    wid = lax.axis_index("s") * 2 + lax.axis_index("c")
    base = wid * CHUNK

    nch = 8
    ich = NIDX // nch
    offs = [lax.rem(jnp.int32(wid // 4 + k), jnp.int32(nch)) * ich
            for k in range(nch)]
    cps = [pltpu.async_copy(idx_hbm.at[pl.ds(offs[k], ich)],
                            idx_v.at[pl.ds(offs[k], ich)], sems[k])
           for k in range(nch)]

    zeros = jnp.zeros((L,), jnp.float32)

    @plsc.parallel_loop(0, CHUNK // L, unroll=8)
    def _zero_body(i):
        chunk_v[pl.ds(i * L, L)] = zeros

    ones = jnp.ones((L,), jnp.float32)

    for k in range(nch):
        cps[k].wait()
        off_k = offs[k]

        @plsc.parallel_loop(0, ich // L, unroll=16)
        def _scan_body(j):
            loc = idx_v[pl.ds(off_k + j * L, L)] - base
            m = plsc.bitcast(loc, jnp.uint32) < jnp.uint32(CHUNK)
            plsc.store_scatter(chunk_v, [loc], ones, mask=m)

    def _wout(total):
        q = total // 4 // 8 * 8
        szs = [q, q, q, total - 3 * q]
        ws = []
        o = 0
        for k in range(4):
            ws.append(pltpu.async_copy(chunk_v.at[pl.ds(o, szs[k])],
                                       out_hbm.at[pl.ds(base + o, szs[k])],
                                       sems[k]))
            o += szs[k]
        for w in ws:
            w.wait()

    @pl.when(wid < NW - 1)
    def _():
        _wout(CHUNK)

    @pl.when(wid == NW - 1)
    def _():
        _wout(LAST)


def kernel(n_range, s, idx):
    del n_range, s
    return (_scatter_ones(idx.astype(jnp.int32)),)


# speedup vs baseline: 1.0821x; 1.0821x over previous
import functools

import jax
import jax.numpy as jnp
from jax import lax
from jax.experimental import pallas as pl
from jax.experimental.pallas import tpu as pltpu
from jax.experimental.pallas import tpu_sc as plsc

N = 1_000_000
NIDX = 65536
NW = 16
CHUNK = 62720
LAST = N - 15 * CHUNK
L = 16


@functools.partial(
    pl.kernel,
    mesh=plsc.VectorSubcoreMesh(core_axis_name="c", subcore_axis_name="s",
                                num_cores=1),
    out_type=jax.ShapeDtypeStruct((N,), jnp.float32),
    scratch_types=[
        pltpu.VMEM((NIDX,), jnp.int32),
        pltpu.VMEM((CHUNK,), jnp.float32),
        [pltpu.SemaphoreType.DMA] * 8,
    ],
    compiler_params=pltpu.CompilerParams(needs_layout_passes=False),
)
def _scatter_ones(idx_hbm, out_hbm, idx_v, chunk_v, sems):
    wid = lax.axis_index("s")
    base = wid * CHUNK

    nch = 8
    ich = NIDX // nch
    offs = [lax.rem(jnp.int32(wid // 2 + k), jnp.int32(nch)) * ich
            for k in range(nch)]
    cps = [pltpu.async_copy(idx_hbm.at[pl.ds(offs[k], ich)],
                            idx_v.at[pl.ds(offs[k], ich)], sems[k])
           for k in range(nch)]

    zeros = jnp.zeros((L,), jnp.float32)

    @plsc.parallel_loop(0, CHUNK // L, unroll=8)
    def _zero_body(i):
        chunk_v[pl.ds(i * L, L)] = zeros

    ones = jnp.ones((L,), jnp.float32)

    for k in range(nch):
        cps[k].wait()
        off_k = offs[k]

        @plsc.parallel_loop(0, ich // L, unroll=16)
        def _scan_body(j):
            loc = idx_v[pl.ds(off_k + j * L, L)] - base
            m = plsc.bitcast(loc, jnp.uint32) < jnp.uint32(CHUNK)
            plsc.store_scatter(chunk_v, [loc], ones, mask=m)

    def _wout(total):
        q = total // 4 // 8 * 8
        szs = [q, q, q, total - 3 * q]
        ws = []
        o = 0
        for k in range(4):
            ws.append(pltpu.async_copy(chunk_v.at[pl.ds(o, szs[k])],
                                       out_hbm.at[pl.ds(base + o, szs[k])],
                                       sems[k]))
            o += szs[k]
        for w in ws:
            w.wait()

    @pl.when(wid < NW - 1)
    def _():
        _wout(CHUNK)

    @pl.when(wid == NW - 1)
    def _():
        _wout(LAST)


def kernel(n_range, s, idx):
    del n_range, s
    return (_scatter_ones(idx.astype(jnp.int32)),)
